# Initial kernel scaffold; baseline (speedup 1.0000x reference)
import jax
import jax.numpy as jnp
from jax.experimental import pallas as pl


def kernel(x, pos_embedding):
    raise NotImplementedError("write your pallas kernel here")


# speedup vs baseline: 1.4321x; 1.4321x over previous
import jax
import jax.numpy as jnp
from jax.experimental import pallas as pl


def _add_body(x_ref, t_ref, o_ref):
    o_ref[...] = x_ref[...] + t_ref[...][None]


def kernel(x, pos_embedding):
    B, D1, D2, d = x.shape
    S = D1 * D2
    xf = x.reshape(B, S, d)
    R = 512
    out = pl.pallas_call(
        _add_body,
        grid=(S // R, B),
        in_specs=[
            pl.BlockSpec((1, R, d), lambda s, b: (b, s, 0)),
            pl.BlockSpec((R, d), lambda s, b: (s, 0)),
        ],
        out_specs=pl.BlockSpec((1, R, d), lambda s, b: (b, s, 0)),
        out_shape=jax.ShapeDtypeStruct((B, S, d), x.dtype),
    )(xf, pos_embedding)
    return out.reshape(B, D1, D2, d)
